# Initial kernel scaffold; baseline (speedup 1.0000x reference)
import jax
import jax.numpy as jnp
from jax.experimental import pallas as pl


def kernel(x, Wg, bg, We, be):
    raise NotImplementedError("write your pallas kernel here")


# speedup vs baseline: 1.3286x; 1.3286x over previous
import functools

import jax
import jax.numpy as jnp
from jax.experimental import pallas as pl
from jax.experimental.pallas import tpu as pltpu

N, D, O, E, TOP_K = 4096, 1024, 1024, 8, 2
BM = 512


def _body(x_ref, wg_ref, bg_ref, we_ref, be_ref,
          pred_ref, logits_ref, sw_ref, ti_ref, tv_ref):
    xb = x_ref[...]
    xb16 = xb.astype(jnp.bfloat16)
    wg16 = wg_ref[...].astype(jnp.bfloat16)
    logits = jnp.dot(xb16, wg16, preferred_element_type=jnp.float32)
    logits = logits + bg_ref[...]
    logits_ref[...] = logits

    m = jnp.max(logits, axis=1, keepdims=True)
    p = jnp.exp(logits - m)
    probs = p / jnp.sum(p, axis=1, keepdims=True)

    iota = jax.lax.broadcasted_iota(jnp.int32, (BM, E), 1)
    i0 = jnp.argmax(probs, axis=1).astype(jnp.int32)
    mask0 = iota == i0[:, None]
    v0 = jnp.max(probs, axis=1)
    masked = jnp.where(mask0, -1.0, probs)
    i1 = jnp.argmax(masked, axis=1).astype(jnp.int32)
    v1 = jnp.max(masked, axis=1)
    mask1 = iota == i1[:, None]

    ti_ref[...] = jnp.concatenate([i0[:, None], i1[:, None]], axis=1)
    tv_ref[...] = jnp.concatenate([v0[:, None], v1[:, None]], axis=1)

    denom = v0 + v1 + 1e-8
    sw = jnp.where(mask0 | mask1, probs, 0.0) / denom[:, None]
    sw_ref[...] = sw

    acc = jnp.dot(sw, be_ref[...], preferred_element_type=jnp.float32)
    for e in range(E):
        we16 = we_ref[e].astype(jnp.bfloat16)
        acc = acc + sw[:, e:e + 1] * jnp.dot(xb16, we16,
                                             preferred_element_type=jnp.float32)
    pred_ref[...] = acc


@jax.jit
def kernel(x, Wg, bg, We, be):
    grid = (N // BM,)
    out = pl.pallas_call(
        _body,
        grid=grid,
        in_specs=[
            pl.BlockSpec((BM, D), lambda i: (i, 0)),
            pl.BlockSpec((D, E), lambda i: (0, 0)),
            pl.BlockSpec((1, E), lambda i: (0, 0)),
            pl.BlockSpec((E, D, O), lambda i: (0, 0, 0)),
            pl.BlockSpec((E, O), lambda i: (0, 0)),
        ],
        out_specs=[
            pl.BlockSpec((BM, O), lambda i: (i, 0)),
            pl.BlockSpec((BM, E), lambda i: (i, 0)),
            pl.BlockSpec((BM, E), lambda i: (i, 0)),
            pl.BlockSpec((BM, TOP_K), lambda i: (i, 0)),
            pl.BlockSpec((BM, TOP_K), lambda i: (i, 0)),
        ],
        out_shape=[
            jax.ShapeDtypeStruct((N, O), jnp.float32),
            jax.ShapeDtypeStruct((N, E), jnp.float32),
            jax.ShapeDtypeStruct((N, E), jnp.float32),
            jax.ShapeDtypeStruct((N, TOP_K), jnp.int32),
            jax.ShapeDtypeStruct((N, TOP_K), jnp.float32),
        ],
        compiler_params=pltpu.CompilerParams(
            dimension_semantics=("arbitrary",),
        ),
    )(x, Wg, bg.reshape(1, E), We, be)
    pred, logits, sw, ti, tv = out
    return (pred, logits, sw, ti, tv)
